# Initial kernel scaffold; baseline (speedup 1.0000x reference)
import jax
import jax.numpy as jnp
from jax.experimental import pallas as pl


def kernel(x, edge_weight, W1, b1, W2, b2, Wp, bp, edge_index):
    raise NotImplementedError("write your pallas kernel here")


# speedup vs baseline: 4.8705x; 4.8705x over previous
import functools

import jax
import jax.numpy as jnp
import numpy as np
from jax import lax
from jax.experimental import pallas as pl
from jax.experimental.pallas import tpu as pltpu
from jax.experimental.pallas import tpu_sc as plsc

_C = 80
_S = 25
_LANES = 16
_PAD = 144


def _perm_host(n: int):
    try:
        cpu = jax.devices("cpu")[0]
        with jax.default_device(cpu):
            return np.asarray(jax.random.permutation(jax.random.key(2025), n))
    except Exception:
        return jax.random.permutation(jax.random.key(2025), n)


def _sc_aggregate(x_ext, src3, dst3, ew3, perm2, n_nodes):
    E_per_tec, nchunk = src3.shape[1] * src3.shape[2], src3.shape[1]
    mesh = plsc.VectorSubcoreMesh(core_axis_name="c", subcore_axis_name="s")
    nblk = n_nodes // _C
    kmax = (nblk + 15) // 16

    @functools.partial(
        pl.kernel,
        out_type=jax.ShapeDtypeStruct((2, n_nodes, _PAD), jnp.float32),
        mesh=mesh,
        scratch_types=[
            pltpu.VMEM((_S, _C), jnp.int32),
            pltpu.VMEM((_S, _C), jnp.int32),
            pltpu.VMEM((_S, _C), jnp.float32),
            pltpu.VMEM((n_nodes,), jnp.int32),
            pltpu.VMEM((_C,), jnp.int32),
            pltpu.VMEM((_C, _PAD), jnp.float32),
            pltpu.VMEM_SHARED((n_nodes, _PAD), jnp.float32),
        ],
        compiler_params=pltpu.CompilerParams(needs_layout_passes=False,
                                             use_tc_tiling_on_sc=False),
    )
    def agg(x_hbm, src_hbm, dst_hbm, ew_hbm, perm_hbm, out_hbm,
            src_v, dst_v, ew_v, perm_v, sidx_v, rows_v, acc_sp):
        cid = lax.axis_index("c")
        sid = lax.axis_index("s")

        pltpu.sync_copy(perm_hbm.at[cid], perm_v)

        def _zrow(r, _):
            for k in range(_PAD // _LANES):
                rows_v[r, pl.ds(k * _LANES, _LANES)] = jnp.zeros(
                    (_LANES,), jnp.float32)
            return 0
        lax.fori_loop(0, _C, _zrow, 0)

        def _zcopy(k, _):
            idx = sid + k * 16

            @pl.when(idx < nblk)
            def _():
                pltpu.sync_copy(rows_v, acc_sp.at[pl.ds(idx * _C, _C)])
            return 0
        lax.fori_loop(0, kmax, _zcopy, 0)
        plsc.subcore_barrier()

        def _super(jo, _):
            pltpu.sync_copy(src_hbm.at[sid].at[pl.ds(jo * _S, _S)], src_v)
            pltpu.sync_copy(dst_hbm.at[sid].at[pl.ds(jo * _S, _S)], dst_v)
            pltpu.sync_copy(ew_hbm.at[sid].at[pl.ds(jo * _S, _S)], ew_v)

            def _chunk(j, _):
                for g in range(_C // _LANES):
                    sv = src_v[j, pl.ds(g * _LANES, _LANES)]
                    sidx_v[pl.ds(g * _LANES, _LANES)] = plsc.load_gather(perm_v, [sv])
                pltpu.sync_copy(x_hbm.at[sidx_v], rows_v)

                def _scale(e, _):
                    w = plsc.load_gather(
                        ew_v, [jnp.full((_LANES,), j, jnp.int32),
                               jnp.full((_LANES,), e, jnp.int32)])
                    for r in range(_PAD // _LANES):
                        v = rows_v[e, pl.ds(r * _LANES, _LANES)]
                        rows_v[e, pl.ds(r * _LANES, _LANES)] = v * w
                    return 0
                lax.fori_loop(0, _C, _scale, 0)

                pltpu.sync_copy(rows_v, acc_sp.at[dst_v.at[j]], add=True)
                return 0
            lax.fori_loop(0, _S, _chunk, 0)
            return 0
        lax.fori_loop(0, nchunk // _S, _super, 0)
        plsc.subcore_barrier()

        def _flush(k, _):
            idx = sid + k * 16

            @pl.when(idx < nblk)
            def _():
                base = idx * _C
                pltpu.sync_copy(acc_sp.at[pl.ds(base, _C)], rows_v)
                pltpu.sync_copy(rows_v, out_hbm.at[cid].at[pl.ds(base, _C)])
            return 0
        lax.fori_loop(0, kmax, _flush, 0)
        return

    return agg(x_ext, src3, dst3, ew3, perm2)


def _tc_dense(acc, W1, b1, W2, b2, Wp, bp, n_nodes, d):
    blk = 1000
    grid = n_nodes // blk

    def body(acc_ref, w1_ref, b1_ref, w2_ref, b2_ref, wp_ref, bp_ref,
             z1_ref, z2_ref, z1n_ref, z2n_ref, g1_ref, g2_ref,
             s1_ref, s2_ref):
        i = pl.program_id(0)

        @pl.when(i == 0)
        def _init():
            s1_ref[...] = jnp.zeros_like(s1_ref)
            s2_ref[...] = jnp.zeros_like(s2_ref)

        def gcn_block(a_slice, w_ref, b_ref):
            feat = a_slice[:, :d]
            deg = a_slice[:, d:d + 1]
            h = feat / jnp.clip(deg, 1e-6, None)
            z = jnp.maximum(
                jnp.dot(h, w_ref[...], preferred_element_type=jnp.float32)
                + b_ref[...][None, :], 0.0)
            return z

        z1 = gcn_block(acc_ref[0], w1_ref, b1_ref)
        z2 = gcn_block(acc_ref[0], w2_ref, b2_ref)
        z1n = gcn_block(acc_ref[1], w1_ref, b1_ref)
        z2n = gcn_block(acc_ref[1], w2_ref, b2_ref)
        z1_ref[...] = z1
        z2_ref[...] = z2
        z1n_ref[...] = z1n
        z2n_ref[...] = z2n
        s1_ref[...] += jnp.sum(z1, axis=0, keepdims=True)
        s2_ref[...] += jnp.sum(z2, axis=0, keepdims=True)

        @pl.when(i == grid - 1)
        def _fin():
            m1 = jax.nn.sigmoid(s1_ref[...] / n_nodes)
            m2 = jax.nn.sigmoid(s2_ref[...] / n_nodes)
            g1_ref[...] = jnp.dot(m1, wp_ref[...],
                                  preferred_element_type=jnp.float32) + bp_ref[...][None, :]
            g2_ref[...] = jnp.dot(m2, wp_ref[...],
                                  preferred_element_type=jnp.float32) + bp_ref[...][None, :]

    zspec = pl.BlockSpec((blk, d), lambda i: (i, 0))
    wspec = pl.BlockSpec((d, d), lambda i: (0, 0))
    bspec = pl.BlockSpec((d,), lambda i: (0,))
    gspec = pl.BlockSpec((1, d), lambda i: (0, 0))
    return pl.pallas_call(
        body,
        grid=(grid,),
        in_specs=[pl.BlockSpec((2, blk, _PAD), lambda i: (0, i, 0)),
                  wspec, bspec, wspec, bspec, wspec, bspec],
        out_specs=[zspec, zspec, zspec, zspec, gspec, gspec],
        out_shape=[jax.ShapeDtypeStruct((n_nodes, d), jnp.float32)] * 4
        + [jax.ShapeDtypeStruct((1, d), jnp.float32)] * 2,
        scratch_shapes=[pltpu.VMEM((1, d), jnp.float32),
                        pltpu.VMEM((1, d), jnp.float32)],
    )(acc, W1, b1, W2, b2, Wp, bp)


def kernel(x, edge_weight, W1, b1, W2, b2, Wp, bp, edge_index):
    n, d = x.shape
    e = edge_index.shape[1]
    per_tec = e // 16
    nchunk = per_tec // _C

    x_ext = jnp.concatenate(
        [x, jnp.ones((n, 1), jnp.float32), jnp.zeros((n, _PAD - d - 1), jnp.float32)],
        axis=1)
    src3 = edge_index[0].reshape(16, nchunk, _C)
    dst3 = edge_index[1].reshape(16, nchunk, _C)
    ew3 = edge_weight.reshape(16, nchunk, _C)
    perm2 = jnp.stack([jnp.arange(n, dtype=jnp.int32),
                       jnp.asarray(_perm_host(n), jnp.int32)])

    acc = _sc_aggregate(x_ext, src3, dst3, ew3, perm2, n)
    z1, z2, z1n, z2n, g1, g2 = _tc_dense(acc, W1, b1, W2, b2, Wp, bp, n, d)
    n_id = jnp.arange(n, dtype=jnp.int32)
    return (z1, z2, g1, g2, z1n, z2n, n_id, 2000)


# speedup vs baseline: 7.5039x; 1.5407x over previous
import functools

import jax
import jax.numpy as jnp
import numpy as np
from jax import lax
from jax.experimental import pallas as pl
from jax.experimental.pallas import tpu as pltpu
from jax.experimental.pallas import tpu_sc as plsc

_C = 80
_S = 25
_LANES = 16
_PAD = 144


def _perm_host(n: int):
    try:
        cpu = jax.devices("cpu")[0]
        with jax.default_device(cpu):
            return np.asarray(jax.random.permutation(jax.random.key(2025), n))
    except Exception:
        return jax.random.permutation(jax.random.key(2025), n)


def _sc_aggregate(x_ext, src3, dst3, ew3, perm2, n_nodes):
    E_per_tec, nchunk = src3.shape[1] * src3.shape[2], src3.shape[1]
    mesh = plsc.VectorSubcoreMesh(core_axis_name="c", subcore_axis_name="s")
    nblk = n_nodes // _C
    kmax = (nblk + 15) // 16

    @functools.partial(
        pl.kernel,
        out_type=jax.ShapeDtypeStruct((2, n_nodes, _PAD), jnp.float32),
        mesh=mesh,
        scratch_types=[
            pltpu.VMEM((_S, _C), jnp.int32),
            pltpu.VMEM((_S, _C), jnp.int32),
            pltpu.VMEM((_S, _C), jnp.float32),
            pltpu.VMEM((n_nodes,), jnp.int32),
            pltpu.VMEM((_C,), jnp.int32),
            pltpu.VMEM((_C,), jnp.int32),
            pltpu.VMEM((_C, _PAD), jnp.float32),
            pltpu.VMEM((_C, _PAD), jnp.float32),
            pltpu.VMEM_SHARED((n_nodes, _PAD), jnp.float32),
            pltpu.SemaphoreType.DMA,
            pltpu.SemaphoreType.DMA,
            pltpu.SemaphoreType.DMA,
            pltpu.SemaphoreType.DMA,
        ],
        compiler_params=pltpu.CompilerParams(needs_layout_passes=False,
                                             use_tc_tiling_on_sc=False),
    )
    def agg(x_hbm, src_hbm, dst_hbm, ew_hbm, perm_hbm, out_hbm,
            src_v, dst_v, ew_v, perm_v, sidx0_v, sidx1_v, rows0_v, rows1_v,
            acc_sp, gsem0, gsem1, ssem0, ssem1):
        sidx = (sidx0_v, sidx1_v)
        rows = (rows0_v, rows1_v)
        gsem = (gsem0, gsem1)
        ssem = (ssem0, ssem1)
        rows_v = rows0_v
        cid = lax.axis_index("c")
        sid = lax.axis_index("s")

        pltpu.sync_copy(perm_hbm.at[cid], perm_v)

        def _zrow(r, _):
            for k in range(_PAD // _LANES):
                rows_v[r, pl.ds(k * _LANES, _LANES)] = jnp.zeros(
                    (_LANES,), jnp.float32)
            return 0
        lax.fori_loop(0, _C, _zrow, 0)

        def _zcopy(k, _):
            idx = sid + k * 16

            @pl.when(idx < nblk)
            def _():
                pltpu.sync_copy(rows_v, acc_sp.at[pl.ds(idx * _C, _C)])
            return 0
        lax.fori_loop(0, kmax, _zcopy, 0)
        plsc.subcore_barrier()

        def _translate(j2, b):
            for g in range(_C // _LANES):
                sv = src_v[j2, pl.ds(g * _LANES, _LANES)]
                sidx[b][pl.ds(g * _LANES, _LANES)] = plsc.load_gather(perm_v, [sv])

        def _scale(j2, b):
            rv = rows[b]

            def _body(t, _):
                for k in range(4):
                    e = t * 4 + k
                    w = plsc.load_gather(
                        ew_v, [jnp.full((_LANES,), j2, jnp.int32),
                               jnp.full((_LANES,), e, jnp.int32)])
                    for r in range(_PAD // _LANES):
                        v = rv[e, pl.ds(r * _LANES, _LANES)]
                        rv[e, pl.ds(r * _LANES, _LANES)] = v * w
                return 0
            lax.fori_loop(0, _C // 4, _body, 0)

        def _super(jo, _):
            pltpu.sync_copy(src_hbm.at[sid].at[pl.ds(jo * _S, _S)], src_v)
            pltpu.sync_copy(dst_hbm.at[sid].at[pl.ds(jo * _S, _S)], dst_v)
            pltpu.sync_copy(ew_hbm.at[sid].at[pl.ds(jo * _S, _S)], ew_v)

            _translate(0, 0)
            pltpu.async_copy(x_hbm.at[sidx[0]], rows[0], gsem[0])
            for j2 in range(_S):
                b = j2 % 2
                if j2 + 1 < _S:
                    if j2 >= 1:
                        pltpu.make_async_copy(
                            rows[1 - b], acc_sp.at[dst_v.at[j2 - 1]],
                            ssem[1 - b]).wait()
                    _translate(j2 + 1, 1 - b)
                    pltpu.async_copy(x_hbm.at[sidx[1 - b]], rows[1 - b],
                                     gsem[1 - b])
                pltpu.make_async_copy(x_hbm.at[sidx[b]], rows[b],
                                      gsem[b]).wait()
                _scale(j2, b)
                pltpu.async_copy(rows[b], acc_sp.at[dst_v.at[j2]], ssem[b],
                                 add=True)
            pltpu.make_async_copy(rows[(_S - 2) % 2],
                                  acc_sp.at[dst_v.at[_S - 2]],
                                  ssem[(_S - 2) % 2]).wait()
            pltpu.make_async_copy(rows[(_S - 1) % 2],
                                  acc_sp.at[dst_v.at[_S - 1]],
                                  ssem[(_S - 1) % 2]).wait()
            return 0
        lax.fori_loop(0, nchunk // _S, _super, 0)
        plsc.subcore_barrier()

        def _flush(k, _):
            idx = sid + k * 16

            @pl.when(idx < nblk)
            def _():
                base = idx * _C
                pltpu.sync_copy(acc_sp.at[pl.ds(base, _C)], rows_v)
                pltpu.sync_copy(rows_v, out_hbm.at[cid].at[pl.ds(base, _C)])
            return 0
        lax.fori_loop(0, kmax, _flush, 0)
        return

    return agg(x_ext, src3, dst3, ew3, perm2)


def _tc_dense(acc, W1, b1, W2, b2, Wp, bp, n_nodes, d):
    blk = 1000
    grid = n_nodes // blk

    def body(acc_ref, w1_ref, b1_ref, w2_ref, b2_ref, wp_ref, bp_ref,
             z1_ref, z2_ref, z1n_ref, z2n_ref, g1_ref, g2_ref,
             s1_ref, s2_ref):
        i = pl.program_id(0)

        @pl.when(i == 0)
        def _init():
            s1_ref[...] = jnp.zeros_like(s1_ref)
            s2_ref[...] = jnp.zeros_like(s2_ref)

        def gcn_block(a_slice, w_ref, b_ref):
            feat = a_slice[:, :d]
            deg = a_slice[:, d:d + 1]
            h = feat / jnp.clip(deg, 1e-6, None)
            z = jnp.maximum(
                jnp.dot(h, w_ref[...], preferred_element_type=jnp.float32)
                + b_ref[...][None, :], 0.0)
            return z

        z1 = gcn_block(acc_ref[0], w1_ref, b1_ref)
        z2 = gcn_block(acc_ref[0], w2_ref, b2_ref)
        z1n = gcn_block(acc_ref[1], w1_ref, b1_ref)
        z2n = gcn_block(acc_ref[1], w2_ref, b2_ref)
        z1_ref[...] = z1
        z2_ref[...] = z2
        z1n_ref[...] = z1n
        z2n_ref[...] = z2n
        s1_ref[...] += jnp.sum(z1, axis=0, keepdims=True)
        s2_ref[...] += jnp.sum(z2, axis=0, keepdims=True)

        @pl.when(i == grid - 1)
        def _fin():
            m1 = jax.nn.sigmoid(s1_ref[...] / n_nodes)
            m2 = jax.nn.sigmoid(s2_ref[...] / n_nodes)
            g1_ref[...] = jnp.dot(m1, wp_ref[...],
                                  preferred_element_type=jnp.float32) + bp_ref[...][None, :]
            g2_ref[...] = jnp.dot(m2, wp_ref[...],
                                  preferred_element_type=jnp.float32) + bp_ref[...][None, :]

    zspec = pl.BlockSpec((blk, d), lambda i: (i, 0))
    wspec = pl.BlockSpec((d, d), lambda i: (0, 0))
    bspec = pl.BlockSpec((d,), lambda i: (0,))
    gspec = pl.BlockSpec((1, d), lambda i: (0, 0))
    return pl.pallas_call(
        body,
        grid=(grid,),
        in_specs=[pl.BlockSpec((2, blk, _PAD), lambda i: (0, i, 0)),
                  wspec, bspec, wspec, bspec, wspec, bspec],
        out_specs=[zspec, zspec, zspec, zspec, gspec, gspec],
        out_shape=[jax.ShapeDtypeStruct((n_nodes, d), jnp.float32)] * 4
        + [jax.ShapeDtypeStruct((1, d), jnp.float32)] * 2,
        scratch_shapes=[pltpu.VMEM((1, d), jnp.float32),
                        pltpu.VMEM((1, d), jnp.float32)],
    )(acc, W1, b1, W2, b2, Wp, bp)


def kernel(x, edge_weight, W1, b1, W2, b2, Wp, bp, edge_index):
    n, d = x.shape
    e = edge_index.shape[1]
    per_tec = e // 16
    nchunk = per_tec // _C

    x_ext = jnp.concatenate(
        [x, jnp.ones((n, 1), jnp.float32), jnp.zeros((n, _PAD - d - 1), jnp.float32)],
        axis=1)
    src3 = edge_index[0].reshape(16, nchunk, _C)
    dst3 = edge_index[1].reshape(16, nchunk, _C)
    ew3 = edge_weight.reshape(16, nchunk, _C)
    perm2 = jnp.stack([jnp.arange(n, dtype=jnp.int32),
                       jnp.asarray(_perm_host(n), jnp.int32)])

    acc = _sc_aggregate(x_ext, src3, dst3, ew3, perm2, n)
    z1, z2, z1n, z2n, g1, g2 = _tc_dense(acc, W1, b1, W2, b2, Wp, bp, n, d)
    n_id = jnp.arange(n, dtype=jnp.int32)
    return (z1, z2, g1, g2, z1n, z2n, n_id, 2000)


# speedup vs baseline: 8.2687x; 1.1019x over previous
import functools

import jax
import jax.numpy as jnp
import numpy as np
from jax import lax
from jax.experimental import pallas as pl
from jax.experimental.pallas import tpu as pltpu
from jax.experimental.pallas import tpu_sc as plsc

_C = 80
_S = 10
_LANES = 16
_PAD = 144


def _perm_host(n: int):
    try:
        cpu = jax.devices("cpu")[0]
        with jax.default_device(cpu):
            return np.asarray(jax.random.permutation(jax.random.key(2025), n))
    except Exception:
        return jax.random.permutation(jax.random.key(2025), n)


def _sc_aggregate(x_ext, src3, dst3, ew3, perm3, n_nodes):
    nchunk = src3.shape[1]
    mesh = plsc.VectorSubcoreMesh(core_axis_name="c", subcore_axis_name="s")
    nblk = n_nodes // _C
    kmax = (nblk + 15) // 16

    @functools.partial(
        pl.kernel,
        out_type=[jax.ShapeDtypeStruct((2, n_nodes, _PAD), jnp.float32),
                  jax.ShapeDtypeStruct((2, n_nodes, _PAD), jnp.float32)],
        mesh=mesh,
        scratch_types=[
            pltpu.VMEM((_S, _C), jnp.int32),
            pltpu.VMEM((_S, _C), jnp.int32),
            pltpu.VMEM((_S, _C), jnp.float32),
            pltpu.VMEM((_C,), jnp.int32),
            pltpu.VMEM((_C, _PAD), jnp.float32),
            pltpu.VMEM((_C, _PAD), jnp.float32),
            pltpu.VMEM((_C, _PAD), jnp.float32),
            pltpu.VMEM_SHARED((n_nodes, _PAD), jnp.float32),
            pltpu.SemaphoreType.DMA,
            pltpu.SemaphoreType.DMA,
            pltpu.SemaphoreType.DMA,
            pltpu.SemaphoreType.DMA,
            pltpu.SemaphoreType.DMA,
            pltpu.SemaphoreType.DMA,
        ],
        compiler_params=pltpu.CompilerParams(needs_layout_passes=False,
                                             use_tc_tiling_on_sc=False),
    )
    def agg(x_hbm, src_hbm, dst_hbm, ew_hbm, perm_hbm, out_hbm, tab_hbm,
            src_v, dst_v, ew_v, pidx_v, rows0_v, rows1_v, rows2_v,
            acc_sp, gsem0, gsem1, gsem2, ssem0, ssem1, ssem2):
        rows = (rows0_v, rows1_v, rows2_v)
        gsem = (gsem0, gsem1, gsem2)
        ssem = (ssem0, ssem1, ssem2)
        cid = lax.axis_index("c")
        sid = lax.axis_index("s")

        @plsc.parallel_loop(0, _C, unroll=4)
        def _zrow(r):
            for k in range(_PAD // _LANES):
                rows0_v[r, pl.ds(k * _LANES, _LANES)] = jnp.zeros(
                    (_LANES,), jnp.float32)

        def _pre(k, _):
            idx = sid + k * 16

            @pl.when(idx < nblk)
            def _():
                base = idx * _C
                pltpu.sync_copy(rows0_v, acc_sp.at[pl.ds(base, _C)])
                pltpu.sync_copy(perm_hbm.at[cid].at[idx], pidx_v)
                pltpu.sync_copy(x_hbm.at[pidx_v], rows1_v)
                pltpu.sync_copy(rows1_v, tab_hbm.at[cid].at[pl.ds(base, _C)])
            return 0
        lax.fori_loop(0, kmax, _pre, 0)
        plsc.subcore_barrier()

        tab = tab_hbm.at[cid]

        def _scale(j2, b):
            rv = rows[b]

            @plsc.parallel_loop(0, _C, unroll=4)
            def _body(e):
                w = plsc.load_gather(
                    ew_v, [jnp.full((_LANES,), j2, jnp.int32),
                           jnp.full((_LANES,), e, jnp.int32)])
                for r in range(_PAD // _LANES):
                    v = rv[e, pl.ds(r * _LANES, _LANES)]
                    rv[e, pl.ds(r * _LANES, _LANES)] = v * w

        def _super(jo, _):
            pltpu.sync_copy(src_hbm.at[sid].at[pl.ds(jo * _S, _S)], src_v)
            pltpu.sync_copy(dst_hbm.at[sid].at[pl.ds(jo * _S, _S)], dst_v)
            pltpu.sync_copy(ew_hbm.at[sid].at[pl.ds(jo * _S, _S)], ew_v)

            pltpu.async_copy(tab.at[src_v.at[0]], rows[0], gsem[0])
            for j2 in range(_S):
                b = j2 % 3
                if j2 + 1 < _S:
                    bn = (j2 + 1) % 3
                    if j2 >= 2:
                        pltpu.make_async_copy(
                            rows[bn], acc_sp.at[dst_v.at[j2 - 2]],
                            ssem[bn]).wait()
                    pltpu.async_copy(tab.at[src_v.at[j2 + 1]], rows[bn],
                                     gsem[bn])
                pltpu.make_async_copy(tab.at[src_v.at[j2]], rows[b],
                                      gsem[b]).wait()
                _scale(j2, b)
                pltpu.async_copy(rows[b], acc_sp.at[dst_v.at[j2]], ssem[b],
                                 add=True)
            for j2 in (_S - 3, _S - 2, _S - 1):
                pltpu.make_async_copy(rows[j2 % 3],
                                      acc_sp.at[dst_v.at[j2]],
                                      ssem[j2 % 3]).wait()
            return 0
        lax.fori_loop(0, nchunk // _S, _super, 0)
        plsc.subcore_barrier()

        def _flush(k, _):
            idx = sid + k * 16

            @pl.when(idx < nblk)
            def _():
                base = idx * _C
                pltpu.sync_copy(acc_sp.at[pl.ds(base, _C)], rows0_v)
                pltpu.sync_copy(rows0_v, out_hbm.at[cid].at[pl.ds(base, _C)])
            return 0
        lax.fori_loop(0, kmax, _flush, 0)
        return

    return agg(x_ext, src3, dst3, ew3, perm3)[0]


def _tc_dense(acc, W1, b1, W2, b2, Wp, bp, n_nodes, d):
    blk = 1000
    grid = n_nodes // blk

    def body(acc_ref, w1_ref, b1_ref, w2_ref, b2_ref, wp_ref, bp_ref,
             z1_ref, z2_ref, z1n_ref, z2n_ref, g1_ref, g2_ref,
             s1_ref, s2_ref):
        i = pl.program_id(0)

        @pl.when(i == 0)
        def _init():
            s1_ref[...] = jnp.zeros_like(s1_ref)
            s2_ref[...] = jnp.zeros_like(s2_ref)

        def gcn_block(a_slice, w_ref, b_ref):
            feat = a_slice[:, :d]
            deg = a_slice[:, d:d + 1]
            h = feat / jnp.clip(deg, 1e-6, None)
            z = jnp.maximum(
                jnp.dot(h, w_ref[...], preferred_element_type=jnp.float32)
                + b_ref[...][None, :], 0.0)
            return z

        z1 = gcn_block(acc_ref[0], w1_ref, b1_ref)
        z2 = gcn_block(acc_ref[0], w2_ref, b2_ref)
        z1n = gcn_block(acc_ref[1], w1_ref, b1_ref)
        z2n = gcn_block(acc_ref[1], w2_ref, b2_ref)
        z1_ref[...] = z1
        z2_ref[...] = z2
        z1n_ref[...] = z1n
        z2n_ref[...] = z2n
        s1_ref[...] += jnp.sum(z1, axis=0, keepdims=True)
        s2_ref[...] += jnp.sum(z2, axis=0, keepdims=True)

        @pl.when(i == grid - 1)
        def _fin():
            m1 = jax.nn.sigmoid(s1_ref[...] / n_nodes)
            m2 = jax.nn.sigmoid(s2_ref[...] / n_nodes)
            g1_ref[...] = jnp.dot(m1, wp_ref[...],
                                  preferred_element_type=jnp.float32) + bp_ref[...][None, :]
            g2_ref[...] = jnp.dot(m2, wp_ref[...],
                                  preferred_element_type=jnp.float32) + bp_ref[...][None, :]

    zspec = pl.BlockSpec((blk, d), lambda i: (i, 0))
    wspec = pl.BlockSpec((d, d), lambda i: (0, 0))
    bspec = pl.BlockSpec((d,), lambda i: (0,))
    gspec = pl.BlockSpec((1, d), lambda i: (0, 0))
    return pl.pallas_call(
        body,
        grid=(grid,),
        in_specs=[pl.BlockSpec((2, blk, _PAD), lambda i: (0, i, 0)),
                  wspec, bspec, wspec, bspec, wspec, bspec],
        out_specs=[zspec, zspec, zspec, zspec, gspec, gspec],
        out_shape=[jax.ShapeDtypeStruct((n_nodes, d), jnp.float32)] * 4
        + [jax.ShapeDtypeStruct((1, d), jnp.float32)] * 2,
        scratch_shapes=[pltpu.VMEM((1, d), jnp.float32),
                        pltpu.VMEM((1, d), jnp.float32)],
    )(acc, W1, b1, W2, b2, Wp, bp)


def kernel(x, edge_weight, W1, b1, W2, b2, Wp, bp, edge_index):
    n, d = x.shape
    e = edge_index.shape[1]
    per_tec = e // 16
    nchunk = per_tec // _C

    x_ext = jnp.concatenate(
        [x, jnp.ones((n, 1), jnp.float32), jnp.zeros((n, _PAD - d - 1), jnp.float32)],
        axis=1)
    src3 = edge_index[0].reshape(16, nchunk, _C)
    dst3 = edge_index[1].reshape(16, nchunk, _C)
    ew3 = edge_weight.reshape(16, nchunk, _C)
    perm3 = jnp.stack([jnp.arange(n, dtype=jnp.int32).reshape(n // _C, _C),
                       jnp.asarray(_perm_host(n), jnp.int32).reshape(n // _C, _C)])

    acc = _sc_aggregate(x_ext, src3, dst3, ew3, perm3, n)
    z1, z2, z1n, z2n, g1, g2 = _tc_dense(acc, W1, b1, W2, b2, Wp, bp, n, d)
    n_id = jnp.arange(n, dtype=jnp.int32)
    return (z1, z2, g1, g2, z1n, z2n, n_id, 2000)


# speedup vs baseline: 8.9810x; 1.0861x over previous
import functools

import jax
import jax.numpy as jnp
import numpy as np
from jax import lax
from jax.experimental import pallas as pl
from jax.experimental.pallas import tpu as pltpu
from jax.experimental.pallas import tpu_sc as plsc

_C = 80
_S = 25
_LANES = 16
_PAD = 144


def _perm_host(n: int):
    try:
        cpu = jax.devices("cpu")[0]
        with jax.default_device(cpu):
            return np.asarray(jax.random.permutation(jax.random.key(2025), n))
    except Exception:
        return jax.random.permutation(jax.random.key(2025), n)


def _sc_aggregate(x_ext, src3, dst3, ew3, perm3, n_nodes):
    nchunk = src3.shape[1]
    mesh = plsc.VectorSubcoreMesh(core_axis_name="c", subcore_axis_name="s")
    nblk = n_nodes // _C
    kmax = (nblk + 15) // 16

    @functools.partial(
        pl.kernel,
        out_type=[jax.ShapeDtypeStruct((2, n_nodes, _PAD), jnp.float32),
                  jax.ShapeDtypeStruct((2, n_nodes, _PAD), jnp.float32)],
        mesh=mesh,
        scratch_types=[
            pltpu.VMEM((_S, _C), jnp.int32),
            pltpu.VMEM((_S, _C), jnp.int32),
            pltpu.VMEM((_S, _C), jnp.float32),
            pltpu.VMEM((_C,), jnp.int32),
            pltpu.VMEM((_C, _PAD), jnp.float32),
            pltpu.VMEM((_C, _PAD), jnp.float32),
            pltpu.VMEM((_C, _PAD), jnp.float32),
            pltpu.VMEM_SHARED((n_nodes, _PAD), jnp.float32),
            pltpu.SemaphoreType.DMA,
            pltpu.SemaphoreType.DMA,
            pltpu.SemaphoreType.DMA,
            pltpu.SemaphoreType.DMA,
            pltpu.SemaphoreType.DMA,
            pltpu.SemaphoreType.DMA,
        ],
        compiler_params=pltpu.CompilerParams(needs_layout_passes=False,
                                             use_tc_tiling_on_sc=False),
    )
    def agg(x_hbm, src_hbm, dst_hbm, ew_hbm, perm_hbm, out_hbm, tab_hbm,
            src_v, dst_v, ew_v, pidx_v, rows0_v, rows1_v, rows2_v,
            acc_sp, gsem0, gsem1, gsem2, ssem0, ssem1, ssem2):
        rows = (rows0_v, rows1_v, rows2_v)
        gsem = (gsem0, gsem1, gsem2)
        ssem = (ssem0, ssem1, ssem2)
        cid = lax.axis_index("c")
        sid = lax.axis_index("s")

        @plsc.parallel_loop(0, _C, unroll=4)
        def _zrow(r):
            for k in range(_PAD // _LANES):
                rows0_v[r, pl.ds(k * _LANES, _LANES)] = jnp.zeros(
                    (_LANES,), jnp.float32)

        def _pre(k, _):
            idx = sid + k * 16

            @pl.when(idx < nblk)
            def _():
                base = idx * _C
                pltpu.sync_copy(rows0_v, acc_sp.at[pl.ds(base, _C)])
                pltpu.sync_copy(perm_hbm.at[cid].at[idx], pidx_v)
                pltpu.sync_copy(x_hbm.at[pidx_v], rows1_v)
                pltpu.sync_copy(rows1_v, tab_hbm.at[cid].at[pl.ds(base, _C)])
            return 0
        lax.fori_loop(0, kmax, _pre, 0)
        plsc.subcore_barrier()

        tab = tab_hbm.at[cid]

        def _scale(j2, b):
            rv = rows[b]

            @plsc.parallel_loop(0, _C, unroll=4)
            def _body(e):
                w = plsc.load_gather(
                    ew_v, [jnp.full((_LANES,), j2, jnp.int32),
                           jnp.full((_LANES,), e, jnp.int32)])
                for r in range(_PAD // _LANES):
                    v = rv[e, pl.ds(r * _LANES, _LANES)]
                    rv[e, pl.ds(r * _LANES, _LANES)] = v * w

        def _super(jo, _):
            pltpu.sync_copy(src_hbm.at[sid].at[pl.ds(jo * _S, _S)], src_v)
            pltpu.sync_copy(dst_hbm.at[sid].at[pl.ds(jo * _S, _S)], dst_v)
            pltpu.sync_copy(ew_hbm.at[sid].at[pl.ds(jo * _S, _S)], ew_v)

            pltpu.async_copy(tab.at[src_v.at[0]], rows[0], gsem[0])
            for j2 in range(_S):
                b = j2 % 3
                if j2 + 1 < _S:
                    bn = (j2 + 1) % 3
                    if j2 >= 2:
                        pltpu.make_async_copy(
                            rows[bn], acc_sp.at[dst_v.at[j2 - 2]],
                            ssem[bn]).wait()
                    pltpu.async_copy(tab.at[src_v.at[j2 + 1]], rows[bn],
                                     gsem[bn])
                pltpu.make_async_copy(tab.at[src_v.at[j2]], rows[b],
                                      gsem[b]).wait()
                _scale(j2, b)
                pltpu.async_copy(rows[b], acc_sp.at[dst_v.at[j2]], ssem[b],
                                 add=True)
            for j2 in (_S - 3, _S - 2, _S - 1):
                pltpu.make_async_copy(rows[j2 % 3],
                                      acc_sp.at[dst_v.at[j2]],
                                      ssem[j2 % 3]).wait()
            return 0
        lax.fori_loop(0, nchunk // _S, _super, 0)
        plsc.subcore_barrier()

        def _flush(k, _):
            idx = sid + k * 16

            @pl.when(idx < nblk)
            def _():
                base = idx * _C
                pltpu.sync_copy(acc_sp.at[pl.ds(base, _C)], rows0_v)
                pltpu.sync_copy(rows0_v, out_hbm.at[cid].at[pl.ds(base, _C)])
            return 0
        lax.fori_loop(0, kmax, _flush, 0)
        return

    return agg(x_ext, src3, dst3, ew3, perm3)[0]


def _tc_dense(acc, W1, b1, W2, b2, Wp, bp, n_nodes, d):
    blk = 1000
    grid = n_nodes // blk

    def body(acc_ref, w1_ref, b1_ref, w2_ref, b2_ref, wp_ref, bp_ref,
             z1_ref, z2_ref, z1n_ref, z2n_ref, g1_ref, g2_ref,
             s1_ref, s2_ref):
        i = pl.program_id(0)

        @pl.when(i == 0)
        def _init():
            s1_ref[...] = jnp.zeros_like(s1_ref)
            s2_ref[...] = jnp.zeros_like(s2_ref)

        def gcn_block(a_slice, w_ref, b_ref):
            feat = a_slice[:, :d]
            deg = a_slice[:, d:d + 1]
            h = feat / jnp.clip(deg, 1e-6, None)
            z = jnp.maximum(
                jnp.dot(h, w_ref[...], preferred_element_type=jnp.float32)
                + b_ref[...][None, :], 0.0)
            return z

        z1 = gcn_block(acc_ref[0], w1_ref, b1_ref)
        z2 = gcn_block(acc_ref[0], w2_ref, b2_ref)
        z1n = gcn_block(acc_ref[1], w1_ref, b1_ref)
        z2n = gcn_block(acc_ref[1], w2_ref, b2_ref)
        z1_ref[...] = z1
        z2_ref[...] = z2
        z1n_ref[...] = z1n
        z2n_ref[...] = z2n
        s1_ref[...] += jnp.sum(z1, axis=0, keepdims=True)
        s2_ref[...] += jnp.sum(z2, axis=0, keepdims=True)

        @pl.when(i == grid - 1)
        def _fin():
            m1 = jax.nn.sigmoid(s1_ref[...] / n_nodes)
            m2 = jax.nn.sigmoid(s2_ref[...] / n_nodes)
            g1_ref[...] = jnp.dot(m1, wp_ref[...],
                                  preferred_element_type=jnp.float32) + bp_ref[...][None, :]
            g2_ref[...] = jnp.dot(m2, wp_ref[...],
                                  preferred_element_type=jnp.float32) + bp_ref[...][None, :]

    zspec = pl.BlockSpec((blk, d), lambda i: (i, 0))
    wspec = pl.BlockSpec((d, d), lambda i: (0, 0))
    bspec = pl.BlockSpec((d,), lambda i: (0,))
    gspec = pl.BlockSpec((1, d), lambda i: (0, 0))
    return pl.pallas_call(
        body,
        grid=(grid,),
        in_specs=[pl.BlockSpec((2, blk, _PAD), lambda i: (0, i, 0)),
                  wspec, bspec, wspec, bspec, wspec, bspec],
        out_specs=[zspec, zspec, zspec, zspec, gspec, gspec],
        out_shape=[jax.ShapeDtypeStruct((n_nodes, d), jnp.float32)] * 4
        + [jax.ShapeDtypeStruct((1, d), jnp.float32)] * 2,
        scratch_shapes=[pltpu.VMEM((1, d), jnp.float32),
                        pltpu.VMEM((1, d), jnp.float32)],
    )(acc, W1, b1, W2, b2, Wp, bp)


def kernel(x, edge_weight, W1, b1, W2, b2, Wp, bp, edge_index):
    n, d = x.shape
    e = edge_index.shape[1]
    per_tec = e // 16
    nchunk = per_tec // _C

    x_ext = jnp.concatenate(
        [x, jnp.ones((n, 1), jnp.float32), jnp.zeros((n, _PAD - d - 1), jnp.float32)],
        axis=1)
    src3 = edge_index[0].reshape(16, nchunk, _C)
    dst3 = edge_index[1].reshape(16, nchunk, _C)
    ew3 = edge_weight.reshape(16, nchunk, _C)
    perm3 = jnp.stack([jnp.arange(n, dtype=jnp.int32).reshape(n // _C, _C),
                       jnp.asarray(_perm_host(n), jnp.int32).reshape(n // _C, _C)])

    acc = _sc_aggregate(x_ext, src3, dst3, ew3, perm3, n)
    z1, z2, z1n, z2n, g1, g2 = _tc_dense(acc, W1, b1, W2, b2, Wp, bp, n, d)
    n_id = jnp.arange(n, dtype=jnp.int32)
    return (z1, z2, g1, g2, z1n, z2n, n_id, 2000)


# speedup vs baseline: 9.9039x; 1.1028x over previous
import functools

import jax
import jax.numpy as jnp
import numpy as np
from jax import lax
from jax.experimental import pallas as pl
from jax.experimental.pallas import tpu as pltpu
from jax.experimental.pallas import tpu_sc as plsc

_C = 80
_S = 25
_LANES = 16
_DW = 16


def _perm_host(n: int):
    try:
        cpu = jax.devices("cpu")[0]
        with jax.default_device(cpu):
            return np.asarray(jax.random.permutation(jax.random.key(2025), n))
    except Exception:
        return jax.random.permutation(jax.random.key(2025), n)


def _sc_aggregate(x, src3, dst3, ew3, perm3, n_nodes, d):
    nchunk = src3.shape[1]
    mesh = plsc.VectorSubcoreMesh(core_axis_name="c", subcore_axis_name="s")
    nblk = n_nodes // _C
    kmax = (nblk + 15) // 16

    @functools.partial(
        pl.kernel,
        out_type=[jax.ShapeDtypeStruct((2, n_nodes, d), jnp.float32),
                  jax.ShapeDtypeStruct((2, n_nodes, _DW), jnp.float32),
                  jax.ShapeDtypeStruct((2, n_nodes, d), jnp.float32)],
        mesh=mesh,
        scratch_types=[
            pltpu.VMEM((_S, _C), jnp.int32),
            pltpu.VMEM((_S, _C), jnp.int32),
            pltpu.VMEM((_S, _C), jnp.float32),
            pltpu.VMEM((_C,), jnp.int32),
            pltpu.VMEM((_C, 128), jnp.float32),
            pltpu.VMEM((_C, 128), jnp.float32),
            pltpu.VMEM((_C, 128), jnp.float32),
            pltpu.VMEM((_C, _DW), jnp.float32),
            pltpu.VMEM((_C, _DW), jnp.float32),
            pltpu.VMEM((_C, _DW), jnp.float32),
            pltpu.VMEM_SHARED((n_nodes, 128), jnp.float32),
            pltpu.VMEM_SHARED((n_nodes, _DW), jnp.float32),
            pltpu.SemaphoreType.DMA,
            pltpu.SemaphoreType.DMA,
            pltpu.SemaphoreType.DMA,
            pltpu.SemaphoreType.DMA,
            pltpu.SemaphoreType.DMA,
            pltpu.SemaphoreType.DMA,
        ],
        compiler_params=pltpu.CompilerParams(needs_layout_passes=False,
                                             use_tc_tiling_on_sc=False),
    )
    def agg(x_hbm, src_hbm, dst_hbm, ew_hbm, perm_hbm,
            out_hbm, deg_hbm, tab_hbm,
            src_v, dst_v, ew_v, pidx_v, rows0_v, rows1_v, rows2_v,
            w0_v, w1_v, w2_v, acc_sp, deg_sp,
            gsem0, gsem1, gsem2, ssem0, ssem1, ssem2):
        rows = (rows0_v, rows1_v, rows2_v)
        wrow = (w0_v, w1_v, w2_v)
        gsem = (gsem0, gsem1, gsem2)
        ssem = (ssem0, ssem1, ssem2)
        cid = lax.axis_index("c")
        sid = lax.axis_index("s")
        lanes_i = lax.iota(jnp.int32, _LANES)

        @plsc.parallel_loop(0, _C, unroll=4)
        def _zrow(r):
            for k in range(128 // _LANES):
                rows0_v[r, pl.ds(k * _LANES, _LANES)] = jnp.zeros(
                    (_LANES,), jnp.float32)
            for wv in wrow:
                wv[r, pl.ds(0, _LANES)] = jnp.zeros((_LANES,), jnp.float32)

        def _pre(k, _):
            idx = sid + k * 16

            @pl.when(idx < nblk)
            def _():
                base = idx * _C
                pltpu.sync_copy(rows0_v, acc_sp.at[pl.ds(base, _C)])
                pltpu.sync_copy(w0_v, deg_sp.at[pl.ds(base, _C)])
                pltpu.sync_copy(perm_hbm.at[cid].at[idx], pidx_v)
                pltpu.sync_copy(x_hbm.at[pidx_v], rows1_v)
                pltpu.sync_copy(rows1_v, tab_hbm.at[cid].at[pl.ds(base, _C)])
            return 0
        lax.fori_loop(0, kmax, _pre, 0)
        plsc.subcore_barrier()

        tab = tab_hbm.at[cid]

        def _scale(j2, b):
            rv = rows[b]
            wv = wrow[b]
            for g in range(_C // _LANES):
                v = ew_v[j2, pl.ds(g * _LANES, _LANES)]
                plsc.store_scatter(wv, [lanes_i + g * _LANES,
                                        jnp.zeros((_LANES,), jnp.int32)], v)

            @plsc.parallel_loop(0, _C, unroll=4)
            def _body(e):
                w = plsc.load_gather(
                    ew_v, [jnp.full((_LANES,), j2, jnp.int32),
                           jnp.full((_LANES,), e, jnp.int32)])
                for r in range(128 // _LANES):
                    v = rv[e, pl.ds(r * _LANES, _LANES)]
                    rv[e, pl.ds(r * _LANES, _LANES)] = v * w

        def _wait_scat(j2):
            b = j2 % 3
            pltpu.make_async_copy(rows[b], acc_sp.at[dst_v.at[j2]],
                                  ssem[b]).wait()
            pltpu.make_async_copy(wrow[b], deg_sp.at[dst_v.at[j2]],
                                  ssem[b]).wait()

        def _super(jo, _):
            pltpu.sync_copy(src_hbm.at[sid].at[pl.ds(jo * _S, _S)], src_v)
            pltpu.sync_copy(dst_hbm.at[sid].at[pl.ds(jo * _S, _S)], dst_v)
            pltpu.sync_copy(ew_hbm.at[sid].at[pl.ds(jo * _S, _S)], ew_v)

            pltpu.async_copy(tab.at[src_v.at[0]], rows[0], gsem[0])
            for j2 in range(_S):
                b = j2 % 3
                if j2 + 1 < _S:
                    bn = (j2 + 1) % 3
                    if j2 >= 2:
                        _wait_scat(j2 - 2)
                    pltpu.async_copy(tab.at[src_v.at[j2 + 1]], rows[bn],
                                     gsem[bn])
                pltpu.make_async_copy(tab.at[src_v.at[j2]], rows[b],
                                      gsem[b]).wait()
                _scale(j2, b)
                pltpu.async_copy(rows[b], acc_sp.at[dst_v.at[j2]], ssem[b],
                                 add=True)
                pltpu.async_copy(wrow[b], deg_sp.at[dst_v.at[j2]], ssem[b],
                                 add=True)
            for j2 in (_S - 3, _S - 2, _S - 1):
                _wait_scat(j2)
            return 0
        lax.fori_loop(0, nchunk // _S, _super, 0)
        plsc.subcore_barrier()

        def _flush(k, _):
            idx = sid + k * 16

            @pl.when(idx < nblk)
            def _():
                base = idx * _C
                pltpu.sync_copy(acc_sp.at[pl.ds(base, _C)], rows0_v)
                pltpu.sync_copy(rows0_v, out_hbm.at[cid].at[pl.ds(base, _C)])
                pltpu.sync_copy(deg_sp.at[pl.ds(base, _C)], w0_v)
                pltpu.sync_copy(w0_v, deg_hbm.at[cid].at[pl.ds(base, _C)])
            return 0
        lax.fori_loop(0, kmax, _flush, 0)
        return

    res = agg(x, src3, dst3, ew3, perm3)
    return res[0], res[1]


def _tc_dense(acc, deg, W1, b1, W2, b2, Wp, bp, n_nodes, d):
    blk = 1000
    grid = n_nodes // blk

    def body(acc_ref, deg_ref, w1_ref, b1_ref, w2_ref, b2_ref, wp_ref, bp_ref,
             z1_ref, z2_ref, z1n_ref, z2n_ref, g1_ref, g2_ref,
             s1_ref, s2_ref):
        i = pl.program_id(0)

        @pl.when(i == 0)
        def _init():
            s1_ref[...] = jnp.zeros_like(s1_ref)
            s2_ref[...] = jnp.zeros_like(s2_ref)

        def gcn_block(a_slice, dg, w_ref, b_ref):
            h = a_slice / jnp.clip(dg, 1e-6, None)
            z = jnp.maximum(
                jnp.dot(h, w_ref[...], preferred_element_type=jnp.float32)
                + b_ref[...][None, :], 0.0)
            return z

        dg0 = deg_ref[0][:, 0:1]
        dg1 = deg_ref[1][:, 0:1]
        z1 = gcn_block(acc_ref[0], dg0, w1_ref, b1_ref)
        z2 = gcn_block(acc_ref[0], dg0, w2_ref, b2_ref)
        z1n = gcn_block(acc_ref[1], dg1, w1_ref, b1_ref)
        z2n = gcn_block(acc_ref[1], dg1, w2_ref, b2_ref)
        z1_ref[...] = z1
        z2_ref[...] = z2
        z1n_ref[...] = z1n
        z2n_ref[...] = z2n
        s1_ref[...] += jnp.sum(z1, axis=0, keepdims=True)
        s2_ref[...] += jnp.sum(z2, axis=0, keepdims=True)

        @pl.when(i == grid - 1)
        def _fin():
            m1 = jax.nn.sigmoid(s1_ref[...] / n_nodes)
            m2 = jax.nn.sigmoid(s2_ref[...] / n_nodes)
            g1_ref[...] = jnp.dot(m1, wp_ref[...],
                                  preferred_element_type=jnp.float32) + bp_ref[...][None, :]
            g2_ref[...] = jnp.dot(m2, wp_ref[...],
                                  preferred_element_type=jnp.float32) + bp_ref[...][None, :]

    zspec = pl.BlockSpec((blk, d), lambda i: (i, 0))
    wspec = pl.BlockSpec((d, d), lambda i: (0, 0))
    bspec = pl.BlockSpec((d,), lambda i: (0,))
    gspec = pl.BlockSpec((1, d), lambda i: (0, 0))
    return pl.pallas_call(
        body,
        grid=(grid,),
        in_specs=[pl.BlockSpec((2, blk, d), lambda i: (0, i, 0)),
                  pl.BlockSpec((2, blk, _DW), lambda i: (0, i, 0)),
                  wspec, bspec, wspec, bspec, wspec, bspec],
        out_specs=[zspec, zspec, zspec, zspec, gspec, gspec],
        out_shape=[jax.ShapeDtypeStruct((n_nodes, d), jnp.float32)] * 4
        + [jax.ShapeDtypeStruct((1, d), jnp.float32)] * 2,
        scratch_shapes=[pltpu.VMEM((1, d), jnp.float32),
                        pltpu.VMEM((1, d), jnp.float32)],
    )(acc, deg, W1, b1, W2, b2, Wp, bp)


def kernel(x, edge_weight, W1, b1, W2, b2, Wp, bp, edge_index):
    n, d = x.shape
    e = edge_index.shape[1]
    per_tec = e // 16
    nchunk = per_tec // _C

    src3 = edge_index[0].reshape(16, nchunk, _C)
    dst3 = edge_index[1].reshape(16, nchunk, _C)
    ew3 = edge_weight.reshape(16, nchunk, _C)
    perm3 = jnp.stack([jnp.arange(n, dtype=jnp.int32).reshape(n // _C, _C),
                       jnp.asarray(_perm_host(n), jnp.int32).reshape(n // _C, _C)])

    acc, deg = _sc_aggregate(x, src3, dst3, ew3, perm3, n, d)
    z1, z2, z1n, z2n, g1, g2 = _tc_dense(acc, deg, W1, b1, W2, b2, Wp, bp, n, d)
    n_id = jnp.arange(n, dtype=jnp.int32)
    return (z1, z2, g1, g2, z1n, z2n, n_id, 2000)


# speedup vs baseline: 11.0508x; 1.1158x over previous
import functools

import jax
import jax.numpy as jnp
import numpy as np
from jax import lax
from jax.experimental import pallas as pl
from jax.experimental.pallas import tpu as pltpu
from jax.experimental.pallas import tpu_sc as plsc

_C = 80
_S = 25
_LANES = 16
_DW = 16


def _perm_host(n: int):
    try:
        cpu = jax.devices("cpu")[0]
        with jax.default_device(cpu):
            return np.asarray(jax.random.permutation(jax.random.key(2025), n))
    except Exception:
        return jax.random.permutation(jax.random.key(2025), n)


def _sc_aggregate(x, src3, dst3, ew3, perm3, n_nodes, d):
    nchunk = src3.shape[1]
    mesh = plsc.VectorSubcoreMesh(core_axis_name="c", subcore_axis_name="s")
    nblk = n_nodes // _C
    kmax = (nblk + 15) // 16

    @functools.partial(
        pl.kernel,
        out_type=[jax.ShapeDtypeStruct((2, n_nodes, d), jnp.float32),
                  jax.ShapeDtypeStruct((2, n_nodes, _DW), jnp.float32),
                  jax.ShapeDtypeStruct((2, n_nodes, d), jnp.bfloat16)],
        mesh=mesh,
        scratch_types=[
            pltpu.VMEM((_S, _C), jnp.int32),
            pltpu.VMEM((_S, _C), jnp.int32),
            pltpu.VMEM((_S, _C), jnp.float32),
            pltpu.VMEM((_C,), jnp.int32),
            pltpu.VMEM((_C, 128), jnp.bfloat16),
            pltpu.VMEM((_C, 128), jnp.bfloat16),
            pltpu.VMEM((_C, 128), jnp.float32),
            pltpu.VMEM((_C, 128), jnp.float32),
            pltpu.VMEM((_C, _DW), jnp.float32),
            pltpu.VMEM((_C, _DW), jnp.float32),
            pltpu.VMEM_SHARED((n_nodes, 128), jnp.float32),
            pltpu.VMEM_SHARED((n_nodes, _DW), jnp.float32),
            pltpu.SemaphoreType.DMA,
            pltpu.SemaphoreType.DMA,
            pltpu.SemaphoreType.DMA,
            pltpu.SemaphoreType.DMA,
        ],
        compiler_params=pltpu.CompilerParams(needs_layout_passes=False,
                                             use_tc_tiling_on_sc=False),
    )
    def agg(x_hbm, src_hbm, dst_hbm, ew_hbm, perm_hbm,
            out_hbm, deg_hbm, tab_hbm,
            src_v, dst_v, ew_v, pidx_v, g0_v, g1_v, f0_v, f1_v,
            w0_v, w1_v, acc_sp, deg_sp,
            gsem0, gsem1, ssem0, ssem1):
        gbuf = (g0_v, g1_v)
        fbuf = (f0_v, f1_v)
        wrow = (w0_v, w1_v)
        gsem = (gsem0, gsem1)
        ssem = (ssem0, ssem1)
        cid = lax.axis_index("c")
        sid = lax.axis_index("s")
        lanes_i = lax.iota(jnp.int32, _LANES)

        @plsc.parallel_loop(0, _C, unroll=4)
        def _zrow(r):
            for k in range(128 // _LANES):
                f0_v[r, pl.ds(k * _LANES, _LANES)] = jnp.zeros(
                    (_LANES,), jnp.float32)
            for wv in wrow:
                wv[r, pl.ds(0, _LANES)] = jnp.zeros((_LANES,), jnp.float32)

        def _pre(k, _):
            idx = sid + k * 16

            @pl.when(idx < nblk)
            def _():
                base = idx * _C
                pltpu.sync_copy(f0_v, acc_sp.at[pl.ds(base, _C)])
                pltpu.sync_copy(w0_v, deg_sp.at[pl.ds(base, _C)])
                pltpu.sync_copy(perm_hbm.at[cid].at[idx], pidx_v)
                pltpu.sync_copy(x_hbm.at[pidx_v], f1_v)

                @plsc.parallel_loop(0, _C, unroll=4)
                def _cv(r):
                    for kk in range(128 // (2 * _LANES)):
                        a = f1_v[r, pl.ds(2 * kk * _LANES, _LANES)]
                        b = f1_v[r, pl.ds((2 * kk + 1) * _LANES, _LANES)]
                        g0_v[r, pl.ds(2 * kk * _LANES, 2 * _LANES)] = (
                            plsc.pack(a, b, format=plsc.PackFormat.INTERLEAVED))
                pltpu.sync_copy(g0_v, tab_hbm.at[cid].at[pl.ds(base, _C)])
            return 0
        lax.fori_loop(0, kmax, _pre, 0)
        plsc.subcore_barrier()

        tab = tab_hbm.at[cid]

        def _scale(j2, b):
            gv = gbuf[b]
            fv = fbuf[b]
            wv = wrow[b]
            for g in range(_C // _LANES):
                v = ew_v[j2, pl.ds(g * _LANES, _LANES)]
                plsc.store_scatter(wv, [lanes_i + g * _LANES,
                                        jnp.zeros((_LANES,), jnp.int32)], v)

            @plsc.parallel_loop(0, _C, unroll=4)
            def _body(e):
                w = plsc.load_gather(
                    ew_v, [jnp.full((_LANES,), j2, jnp.int32),
                           jnp.full((_LANES,), e, jnp.int32)])
                for r in range(128 // (2 * _LANES)):
                    ab = gv[e, pl.ds(2 * r * _LANES, 2 * _LANES)]
                    a, b2 = plsc.unpack(ab, format=plsc.PackFormat.INTERLEAVED)
                    fv[e, pl.ds(2 * r * _LANES, _LANES)] = a * w
                    fv[e, pl.ds((2 * r + 1) * _LANES, _LANES)] = b2 * w

        def _wait_scat(j2):
            b = j2 % 2
            pltpu.make_async_copy(fbuf[b], acc_sp.at[dst_v.at[j2]],
                                  ssem[b]).wait()
            pltpu.make_async_copy(wrow[b], deg_sp.at[dst_v.at[j2]],
                                  ssem[b]).wait()

        def _super(jo, _):
            pltpu.sync_copy(src_hbm.at[sid].at[pl.ds(jo * _S, _S)], src_v)
            pltpu.sync_copy(dst_hbm.at[sid].at[pl.ds(jo * _S, _S)], dst_v)
            pltpu.sync_copy(ew_hbm.at[sid].at[pl.ds(jo * _S, _S)], ew_v)

            pltpu.async_copy(tab.at[src_v.at[0]], gbuf[0], gsem[0])
            for j2 in range(_S):
                b = j2 % 2
                if j2 + 1 < _S:
                    pltpu.async_copy(tab.at[src_v.at[j2 + 1]], gbuf[1 - b],
                                     gsem[1 - b])
                pltpu.make_async_copy(tab.at[src_v.at[j2]], gbuf[b],
                                      gsem[b]).wait()
                if j2 >= 2:
                    _wait_scat(j2 - 2)
                _scale(j2, b)
                pltpu.async_copy(fbuf[b], acc_sp.at[dst_v.at[j2]], ssem[b],
                                 add=True)
                pltpu.async_copy(wrow[b], deg_sp.at[dst_v.at[j2]], ssem[b],
                                 add=True)
            for j2 in (_S - 2, _S - 1):
                _wait_scat(j2)
            return 0
        lax.fori_loop(0, nchunk // _S, _super, 0)
        plsc.subcore_barrier()

        def _flush(k, _):
            idx = sid + k * 16

            @pl.when(idx < nblk)
            def _():
                base = idx * _C
                pltpu.sync_copy(acc_sp.at[pl.ds(base, _C)], f0_v)
                pltpu.sync_copy(f0_v, out_hbm.at[cid].at[pl.ds(base, _C)])
                pltpu.sync_copy(deg_sp.at[pl.ds(base, _C)], w0_v)
                pltpu.sync_copy(w0_v, deg_hbm.at[cid].at[pl.ds(base, _C)])
            return 0
        lax.fori_loop(0, kmax, _flush, 0)
        return

    res = agg(x, src3, dst3, ew3, perm3)
    return res[0], res[1]


def _tc_dense(acc, deg, W1, b1, W2, b2, Wp, bp, n_nodes, d):
    blk = 1000
    grid = n_nodes // blk

    def body(acc_ref, deg_ref, w1_ref, b1_ref, w2_ref, b2_ref, wp_ref, bp_ref,
             z1_ref, z2_ref, z1n_ref, z2n_ref, g1_ref, g2_ref,
             s1_ref, s2_ref):
        i = pl.program_id(0)

        @pl.when(i == 0)
        def _init():
            s1_ref[...] = jnp.zeros_like(s1_ref)
            s2_ref[...] = jnp.zeros_like(s2_ref)

        def gcn_block(a_slice, dg, w_ref, b_ref):
            h = a_slice / jnp.clip(dg, 1e-6, None)
            z = jnp.maximum(
                jnp.dot(h, w_ref[...], preferred_element_type=jnp.float32)
                + b_ref[...][None, :], 0.0)
            return z

        dg0 = deg_ref[0][:, 0:1]
        dg1 = deg_ref[1][:, 0:1]
        z1 = gcn_block(acc_ref[0], dg0, w1_ref, b1_ref)
        z2 = gcn_block(acc_ref[0], dg0, w2_ref, b2_ref)
        z1n = gcn_block(acc_ref[1], dg1, w1_ref, b1_ref)
        z2n = gcn_block(acc_ref[1], dg1, w2_ref, b2_ref)
        z1_ref[...] = z1
        z2_ref[...] = z2
        z1n_ref[...] = z1n
        z2n_ref[...] = z2n
        s1_ref[...] += jnp.sum(z1, axis=0, keepdims=True)
        s2_ref[...] += jnp.sum(z2, axis=0, keepdims=True)

        @pl.when(i == grid - 1)
        def _fin():
            m1 = jax.nn.sigmoid(s1_ref[...] / n_nodes)
            m2 = jax.nn.sigmoid(s2_ref[...] / n_nodes)
            g1_ref[...] = jnp.dot(m1, wp_ref[...],
                                  preferred_element_type=jnp.float32) + bp_ref[...][None, :]
            g2_ref[...] = jnp.dot(m2, wp_ref[...],
                                  preferred_element_type=jnp.float32) + bp_ref[...][None, :]

    zspec = pl.BlockSpec((blk, d), lambda i: (i, 0))
    wspec = pl.BlockSpec((d, d), lambda i: (0, 0))
    bspec = pl.BlockSpec((d,), lambda i: (0,))
    gspec = pl.BlockSpec((1, d), lambda i: (0, 0))
    return pl.pallas_call(
        body,
        grid=(grid,),
        in_specs=[pl.BlockSpec((2, blk, d), lambda i: (0, i, 0)),
                  pl.BlockSpec((2, blk, _DW), lambda i: (0, i, 0)),
                  wspec, bspec, wspec, bspec, wspec, bspec],
        out_specs=[zspec, zspec, zspec, zspec, gspec, gspec],
        out_shape=[jax.ShapeDtypeStruct((n_nodes, d), jnp.float32)] * 4
        + [jax.ShapeDtypeStruct((1, d), jnp.float32)] * 2,
        scratch_shapes=[pltpu.VMEM((1, d), jnp.float32),
                        pltpu.VMEM((1, d), jnp.float32)],
    )(acc, deg, W1, b1, W2, b2, Wp, bp)


def kernel(x, edge_weight, W1, b1, W2, b2, Wp, bp, edge_index):
    n, d = x.shape
    e = edge_index.shape[1]
    per_tec = e // 16
    nchunk = per_tec // _C

    src3 = edge_index[0].reshape(16, nchunk, _C)
    dst3 = edge_index[1].reshape(16, nchunk, _C)
    ew3 = edge_weight.reshape(16, nchunk, _C)
    perm3 = jnp.stack([jnp.arange(n, dtype=jnp.int32).reshape(n // _C, _C),
                       jnp.asarray(_perm_host(n), jnp.int32).reshape(n // _C, _C)])

    acc, deg = _sc_aggregate(x, src3, dst3, ew3, perm3, n, d)
    z1, z2, z1n, z2n, g1, g2 = _tc_dense(acc, deg, W1, b1, W2, b2, Wp, bp, n, d)
    n_id = jnp.arange(n, dtype=jnp.int32)
    return (z1, z2, g1, g2, z1n, z2n, n_id, 2000)


# speedup vs baseline: 11.1600x; 1.0099x over previous
import functools

import jax
import jax.numpy as jnp
import numpy as np
from jax import lax
from jax.experimental import pallas as pl
from jax.experimental.pallas import tpu as pltpu
from jax.experimental.pallas import tpu_sc as plsc

_C = 80
_S = 25
_LANES = 16
_DW = 16


def _perm_host(n: int):
    try:
        cpu = jax.devices("cpu")[0]
        with jax.default_device(cpu):
            return np.asarray(jax.random.permutation(jax.random.key(2025), n))
    except Exception:
        return jax.random.permutation(jax.random.key(2025), n)


def _sc_aggregate(x, src3, dst3, ew3, perm3, n_nodes, d):
    nchunk = src3.shape[1]
    mesh = plsc.VectorSubcoreMesh(core_axis_name="c", subcore_axis_name="s")
    nblk = n_nodes // _C
    kmax = (nblk + 15) // 16

    @functools.partial(
        pl.kernel,
        out_type=[jax.ShapeDtypeStruct((2, n_nodes, d), jnp.float32),
                  jax.ShapeDtypeStruct((2, n_nodes, _DW), jnp.float32),
                  jax.ShapeDtypeStruct((2, n_nodes, d), jnp.bfloat16)],
        mesh=mesh,
        scratch_types=[
            pltpu.VMEM((_S, _C), jnp.int32),
            pltpu.VMEM((_S, _C), jnp.int32),
            pltpu.VMEM((_S, _C), jnp.float32),
            pltpu.VMEM((_C,), jnp.int32),
            pltpu.VMEM((_C, 128), jnp.bfloat16),
            pltpu.VMEM((_C, 128), jnp.bfloat16),
            pltpu.VMEM((_C, 128), jnp.float32),
            pltpu.VMEM((_C, 128), jnp.float32),
            pltpu.VMEM((_C, _DW), jnp.float32),
            pltpu.VMEM((_C, _DW), jnp.float32),
            pltpu.VMEM_SHARED((n_nodes, 128), jnp.float32),
            pltpu.VMEM_SHARED((n_nodes, _DW), jnp.float32),
            pltpu.SemaphoreType.DMA,
            pltpu.SemaphoreType.DMA,
            pltpu.SemaphoreType.DMA,
            pltpu.SemaphoreType.DMA,
        ],
        compiler_params=pltpu.CompilerParams(needs_layout_passes=False,
                                             use_tc_tiling_on_sc=False),
    )
    def agg(x_hbm, src_hbm, dst_hbm, ew_hbm, perm_hbm,
            out_hbm, deg_hbm, tab_hbm,
            src_v, dst_v, ew_v, pidx_v, g0_v, g1_v, f0_v, f1_v,
            w0_v, w1_v, acc_sp, deg_sp,
            gsem0, gsem1, ssem0, ssem1):
        gbuf = (g0_v, g1_v)
        fbuf = (f0_v, f1_v)
        wrow = (w0_v, w1_v)
        gsem = (gsem0, gsem1)
        ssem = (ssem0, ssem1)
        cid = lax.axis_index("c")
        sid = lax.axis_index("s")
        lanes_i = lax.iota(jnp.int32, _LANES)

        @plsc.parallel_loop(0, _C, unroll=4)
        def _zrow(r):
            for k in range(128 // _LANES):
                f0_v[r, pl.ds(k * _LANES, _LANES)] = jnp.zeros(
                    (_LANES,), jnp.float32)
            for wv in wrow:
                wv[r, pl.ds(0, _LANES)] = jnp.zeros((_LANES,), jnp.float32)

        def _tab_wait(k):
            @pl.when(sid + k * 16 < nblk)
            def _():
                pb = (sid + k * 16) * _C
                pltpu.make_async_copy(
                    gbuf[k % 2], tab_hbm.at[cid].at[pl.ds(pb, _C)],
                    gsem[k % 2]).wait()

        for k in range(kmax):
            idx = sid + k * 16
            if k >= 2:
                _tab_wait(k - 2)

            @pl.when(idx < nblk)
            def _(k=k, idx=idx):
                base = idx * _C
                pltpu.sync_copy(f0_v, acc_sp.at[pl.ds(base, _C)])
                pltpu.sync_copy(w0_v, deg_sp.at[pl.ds(base, _C)])
                pltpu.sync_copy(perm_hbm.at[cid].at[idx], pidx_v)
                pltpu.sync_copy(x_hbm.at[pidx_v], f1_v)
                gv = gbuf[k % 2]

                @plsc.parallel_loop(0, _C, unroll=4)
                def _cv(r):
                    for kk in range(128 // (2 * _LANES)):
                        a = f1_v[r, pl.ds(2 * kk * _LANES, _LANES)]
                        b = f1_v[r, pl.ds((2 * kk + 1) * _LANES, _LANES)]
                        gv[r, pl.ds(2 * kk * _LANES, 2 * _LANES)] = (
                            plsc.pack(a, b, format=plsc.PackFormat.INTERLEAVED))
                pltpu.async_copy(gv, tab_hbm.at[cid].at[pl.ds(base, _C)],
                                 gsem[k % 2])
        for k in (kmax - 2, kmax - 1):
            _tab_wait(k)
        plsc.subcore_barrier()

        tab = tab_hbm.at[cid]

        def _scale(j2, b):
            gv = gbuf[b]
            fv = fbuf[b]
            wv = wrow[b]
            for g in range(_C // _LANES):
                v = ew_v[j2, pl.ds(g * _LANES, _LANES)]
                plsc.store_scatter(wv, [lanes_i + g * _LANES,
                                        jnp.zeros((_LANES,), jnp.int32)], v)

            @plsc.parallel_loop(0, _C, unroll=4)
            def _body(e):
                w = plsc.load_gather(
                    ew_v, [jnp.full((_LANES,), j2, jnp.int32),
                           jnp.full((_LANES,), e, jnp.int32)])
                for r in range(128 // (2 * _LANES)):
                    ab = gv[e, pl.ds(2 * r * _LANES, 2 * _LANES)]
                    a, b2 = plsc.unpack(ab, format=plsc.PackFormat.INTERLEAVED)
                    fv[e, pl.ds(2 * r * _LANES, _LANES)] = a * w
                    fv[e, pl.ds((2 * r + 1) * _LANES, _LANES)] = b2 * w

        def _wait_scat(j2):
            b = j2 % 2
            pltpu.make_async_copy(fbuf[b], acc_sp.at[dst_v.at[j2]],
                                  ssem[b]).wait()
            pltpu.make_async_copy(wrow[b], deg_sp.at[dst_v.at[j2]],
                                  ssem[b]).wait()

        def _super(jo, _):
            pltpu.sync_copy(src_hbm.at[sid].at[pl.ds(jo * _S, _S)], src_v)
            pltpu.sync_copy(dst_hbm.at[sid].at[pl.ds(jo * _S, _S)], dst_v)
            pltpu.sync_copy(ew_hbm.at[sid].at[pl.ds(jo * _S, _S)], ew_v)

            pltpu.async_copy(tab.at[src_v.at[0]], gbuf[0], gsem[0])
            for j2 in range(_S):
                b = j2 % 2
                if j2 + 1 < _S:
                    pltpu.async_copy(tab.at[src_v.at[j2 + 1]], gbuf[1 - b],
                                     gsem[1 - b])
                pltpu.make_async_copy(tab.at[src_v.at[j2]], gbuf[b],
                                      gsem[b]).wait()
                if j2 >= 2:
                    _wait_scat(j2 - 2)
                _scale(j2, b)
                pltpu.async_copy(fbuf[b], acc_sp.at[dst_v.at[j2]], ssem[b],
                                 add=True)
                pltpu.async_copy(wrow[b], deg_sp.at[dst_v.at[j2]], ssem[b],
                                 add=True)
            for j2 in (_S - 2, _S - 1):
                _wait_scat(j2)
            return 0
        lax.fori_loop(0, nchunk // _S, _super, 0)
        plsc.subcore_barrier()

        def _out_wait(k):
            @pl.when(sid + k * 16 < nblk)
            def _():
                pb = (sid + k * 16) * _C
                pltpu.make_async_copy(
                    fbuf[k % 2], out_hbm.at[cid].at[pl.ds(pb, _C)],
                    gsem[k % 2]).wait()
                pltpu.make_async_copy(
                    wrow[k % 2], deg_hbm.at[cid].at[pl.ds(pb, _C)],
                    ssem[k % 2]).wait()

        for k in range(kmax):
            idx = sid + k * 16
            if k >= 2:
                _out_wait(k - 2)

            @pl.when(idx < nblk)
            def _(k=k, idx=idx):
                base = idx * _C
                pltpu.sync_copy(acc_sp.at[pl.ds(base, _C)], fbuf[k % 2])
                pltpu.sync_copy(deg_sp.at[pl.ds(base, _C)], wrow[k % 2])
                pltpu.async_copy(fbuf[k % 2],
                                 out_hbm.at[cid].at[pl.ds(base, _C)],
                                 gsem[k % 2])
                pltpu.async_copy(wrow[k % 2],
                                 deg_hbm.at[cid].at[pl.ds(base, _C)],
                                 ssem[k % 2])
        for k in (kmax - 2, kmax - 1):
            _out_wait(k)
        return

    res = agg(x, src3, dst3, ew3, perm3)
    return res[0], res[1]


def _tc_dense(acc, deg, W1, b1, W2, b2, Wp, bp, n_nodes, d):
    blk = 1000
    grid = n_nodes // blk

    def body(acc_ref, deg_ref, w1_ref, b1_ref, w2_ref, b2_ref, wp_ref, bp_ref,
             z1_ref, z2_ref, z1n_ref, z2n_ref, g1_ref, g2_ref,
             s1_ref, s2_ref):
        i = pl.program_id(0)

        @pl.when(i == 0)
        def _init():
            s1_ref[...] = jnp.zeros_like(s1_ref)
            s2_ref[...] = jnp.zeros_like(s2_ref)

        def gcn_block(a_slice, dg, w_ref, b_ref):
            h = a_slice / jnp.clip(dg, 1e-6, None)
            z = jnp.maximum(
                jnp.dot(h, w_ref[...], preferred_element_type=jnp.float32)
                + b_ref[...][None, :], 0.0)
            return z

        dg0 = deg_ref[0][:, 0:1]
        dg1 = deg_ref[1][:, 0:1]
        z1 = gcn_block(acc_ref[0], dg0, w1_ref, b1_ref)
        z2 = gcn_block(acc_ref[0], dg0, w2_ref, b2_ref)
        z1n = gcn_block(acc_ref[1], dg1, w1_ref, b1_ref)
        z2n = gcn_block(acc_ref[1], dg1, w2_ref, b2_ref)
        z1_ref[...] = z1
        z2_ref[...] = z2
        z1n_ref[...] = z1n
        z2n_ref[...] = z2n
        s1_ref[...] += jnp.sum(z1, axis=0, keepdims=True)
        s2_ref[...] += jnp.sum(z2, axis=0, keepdims=True)

        @pl.when(i == grid - 1)
        def _fin():
            m1 = jax.nn.sigmoid(s1_ref[...] / n_nodes)
            m2 = jax.nn.sigmoid(s2_ref[...] / n_nodes)
            g1_ref[...] = jnp.dot(m1, wp_ref[...],
                                  preferred_element_type=jnp.float32) + bp_ref[...][None, :]
            g2_ref[...] = jnp.dot(m2, wp_ref[...],
                                  preferred_element_type=jnp.float32) + bp_ref[...][None, :]

    zspec = pl.BlockSpec((blk, d), lambda i: (i, 0))
    wspec = pl.BlockSpec((d, d), lambda i: (0, 0))
    bspec = pl.BlockSpec((d,), lambda i: (0,))
    gspec = pl.BlockSpec((1, d), lambda i: (0, 0))
    return pl.pallas_call(
        body,
        grid=(grid,),
        in_specs=[pl.BlockSpec((2, blk, d), lambda i: (0, i, 0)),
                  pl.BlockSpec((2, blk, _DW), lambda i: (0, i, 0)),
                  wspec, bspec, wspec, bspec, wspec, bspec],
        out_specs=[zspec, zspec, zspec, zspec, gspec, gspec],
        out_shape=[jax.ShapeDtypeStruct((n_nodes, d), jnp.float32)] * 4
        + [jax.ShapeDtypeStruct((1, d), jnp.float32)] * 2,
        scratch_shapes=[pltpu.VMEM((1, d), jnp.float32),
                        pltpu.VMEM((1, d), jnp.float32)],
    )(acc, deg, W1, b1, W2, b2, Wp, bp)


def kernel(x, edge_weight, W1, b1, W2, b2, Wp, bp, edge_index):
    n, d = x.shape
    e = edge_index.shape[1]
    per_tec = e // 16
    nchunk = per_tec // _C

    src3 = edge_index[0].reshape(16, nchunk, _C)
    dst3 = edge_index[1].reshape(16, nchunk, _C)
    ew3 = edge_weight.reshape(16, nchunk, _C)
    perm3 = jnp.stack([jnp.arange(n, dtype=jnp.int32).reshape(n // _C, _C),
                       jnp.asarray(_perm_host(n), jnp.int32).reshape(n // _C, _C)])

    acc, deg = _sc_aggregate(x, src3, dst3, ew3, perm3, n, d)
    z1, z2, z1n, z2n, g1, g2 = _tc_dense(acc, deg, W1, b1, W2, b2, Wp, bp, n, d)
    n_id = jnp.arange(n, dtype=jnp.int32)
    return (z1, z2, g1, g2, z1n, z2n, n_id, 2000)
